# baseline (speedup 1.0000x reference)
import functools

import jax
import jax.numpy as jnp
from jax.experimental import pallas as pl

KNN_K = 32
N_EIGENS = 64
FM_LAMBDA = 1e-3
N = 2048
D = 512
BLK = 256


def _wl_xla(vals, idx, temp):
    w = jax.nn.softmax(vals / temp, axis=-1)
    rows = jnp.broadcast_to(jnp.arange(N)[:, None], (N, KNN_K))
    W = jnp.zeros((N, N), dtype=jnp.float32).at[
        rows.reshape(-1), idx.reshape(-1)].add(w.reshape(-1))
    W = 0.5 * (W + W.T)
    deg = jnp.sum(W, axis=1) + 1e-6
    dinv = jax.lax.rsqrt(deg)
    L = jnp.eye(N, dtype=W.dtype) - dinv[:, None] * W * dinv[None, :]
    return 0.5 * (L + L.T)


def _fm_kernel(vip_ref, tip_ref, vl2_ref, tl2_ref, cxy_ref, cyx_ref):
    vip = vip_ref[...]
    tip = tip_ref[...]
    vl2 = vl2_ref[...]
    tl2 = tl2_ref[...]

    def dot_nt(a, b):
        return jax.lax.dot_general(
            a, b, dimension_numbers=(((1,), (1,)), ((), ())),
            preferred_element_type=jnp.float32)

    def mm(a, b):
        return jax.lax.dot_general(
            a, b, dimension_numbers=(((1,), (0,)), ((), ())),
            preferred_element_type=jnp.float32,
            precision=jax.lax.Precision.HIGHEST)

    eye = jnp.eye(NE_, dtype=jnp.float32)

    def solve_pair(Vx_ip, Vy_ip, Vx_l2, Vy_l2):
        A = dot_nt(Vx_ip, Vx_ip) + dot_nt(Vx_l2, Vx_l2) + FM_LAMBDA * eye
        B = dot_nt(Vy_ip, Vx_ip) + dot_nt(Vy_l2, Vx_l2)
        X = eye * (1.0 / (2.0 + FM_LAMBDA))
        for _ in range(6):
            AX = mm(A, X)
            X = mm(X, 2.0 * eye - AX)
        return mm(B, X)

    cxy_ref[...] = solve_pair(vip, tip, vl2, tl2)
    cyx_ref[...] = solve_pair(tip, vip, tl2, vl2)


NE_ = N_EIGENS


def _pallas_fm(v_ip, t_ip, v_l2, t_l2):
    return pl.pallas_call(
        _fm_kernel,
        in_specs=[pl.BlockSpec((NE_, N), lambda: (0, 0))] * 4,
        out_specs=[pl.BlockSpec((NE_, NE_), lambda: (0, 0))] * 2,
        out_shape=[jax.ShapeDtypeStruct((NE_, NE_), jnp.float32)] * 2,
    )(v_ip, t_ip, v_l2, t_l2)


def _graph_laplacian(G, sqr, sqc, metric):
    if metric == "ip":
        sim = G
    else:
        sim = -(sqr + sqc - 2.0 * G)
    sim = sim - 1e9 * jnp.eye(N, dtype=sim.dtype)
    vals, idx = jax.lax.top_k(sim, KNN_K)
    temp = jnp.std(vals) + 1e-6
    return _wl_xla(vals, idx, temp)


def _gram_kernel(a_ref, b_ref, out_ref):
    out_ref[...] = jax.lax.dot_general(
        a_ref[...], b_ref[...],
        dimension_numbers=(((1,), (1,)), ((), ())),
        preferred_element_type=jnp.float32)


def _gram(feat):
    return pl.pallas_call(
        _gram_kernel,
        grid=(N // BLK,),
        in_specs=[
            pl.BlockSpec((BLK, D), lambda i: (i, 0)),
            pl.BlockSpec((N, D), lambda i: (0, 0)),
        ],
        out_specs=pl.BlockSpec((BLK, N), lambda i: (i, 0)),
        out_shape=jax.ShapeDtypeStruct((N, N), jnp.float32),
    )(feat, feat)


def kernel(feat_t, feat_v):
    G_t = _gram(feat_t)
    G_v = _gram(feat_v)
    sq_t = jnp.sum(feat_t * feat_t, axis=1)
    sq_v = jnp.sum(feat_v * feat_v, axis=1)

    L_vip = _graph_laplacian(G_v, sq_v[:, None], sq_v[None, :], "ip")
    L_tip = _graph_laplacian(G_t, sq_t[:, None], sq_t[None, :], "ip")
    L_vl2 = _graph_laplacian(G_v, sq_v[:, None], sq_v[None, :], "l2")
    L_tl2 = _graph_laplacian(G_t, sq_t[:, None], sq_t[None, :], "l2")

    _, V_vip = jnp.linalg.eigh(L_vip)
    _, V_tip = jnp.linalg.eigh(L_tip)
    _, V_vl2 = jnp.linalg.eigh(L_vl2)
    _, V_tl2 = jnp.linalg.eigh(L_tl2)

    v_ip = V_vip.T[:N_EIGENS, :]
    t_ip = V_tip.T[:N_EIGENS, :]
    v_l2 = V_vl2.T[:N_EIGENS, :]
    t_l2 = V_tl2.T[:N_EIGENS, :]

    Cxy = _fm_solve_xla(v_ip, t_ip, v_l2, t_l2)
    Cyx = _fm_solve_xla(t_ip, v_ip, t_l2, v_l2)
    return (Cxy, Cyx, v_ip, t_ip)


def _fm_solve_xla(Vx_ip, Vy_ip, Vx_l2, Vy_l2):
    k = Vx_ip.shape[0]
    A = Vx_ip @ Vx_ip.T + Vx_l2 @ Vx_l2.T + FM_LAMBDA * jnp.eye(k, dtype=Vx_ip.dtype)
    B = Vy_ip @ Vx_ip.T + Vy_l2 @ Vx_l2.T
    return jnp.linalg.solve(A, B.T).T
